# baseline (speedup 1.0000x reference)
import functools

import jax
import jax.numpy as jnp
from jax import lax
from jax.experimental import pallas as pl
from jax.experimental.pallas import tpu as pltpu
from jax.experimental.pallas import tpu_sc as plsc

B = 1024
L = 200
E = 64
EP = 128
H = 80
HP = 128
G4 = 4 * HP
NCLS = 15
F32 = jnp.float32


@functools.lru_cache(maxsize=None)
def _make_gather():
  info = plsc.get_sparse_core_info()
  nw = info.num_cores * info.num_subcores
  tot = B * L
  per_w = tot // nw
  ch = 800
  n_ch = per_w // ch

  @functools.partial(
      pl.kernel,
      out_type=jax.ShapeDtypeStruct((tot, EP), F32),
      mesh=plsc.VectorSubcoreMesh(core_axis_name="c", subcore_axis_name="s"),
      scratch_types=[
          pltpu.VMEM((ch,), jnp.int32),
          pltpu.VMEM((ch, EP), F32),
          pltpu.SemaphoreType.DMA,
      ],
  )
  def gather_k(table_hbm, idx_hbm, out_hbm, idx_v, buf, sem):
    wid = lax.axis_index("s") * info.num_cores + lax.axis_index("c")
    base = wid * per_w
    for j in range(n_ch):
      off = base + j * ch
      pltpu.sync_copy(idx_hbm.at[pl.ds(off, ch)], idx_v)
      pltpu.async_copy(table_hbm.at[idx_v], buf, sem).wait()
      pltpu.sync_copy(buf, out_hbm.at[pl.ds(off, ch)])

  return gather_k


def _lstm_step(x_t, h_prev, c_prev, wih_ref, whh_ref, b_ref):
  g = jnp.dot(x_t, wih_ref[...], preferred_element_type=F32)
  g = g + jnp.dot(h_prev, whh_ref[...], preferred_element_type=F32)
  g = g + b_ref[...]
  ig = jax.nn.sigmoid(g[:, 0:HP])
  fg = jax.nn.sigmoid(g[:, HP:2 * HP])
  gg = jnp.tanh(g[:, 2 * HP:3 * HP])
  og = jax.nn.sigmoid(g[:, 3 * HP:4 * HP])
  c_new = fg * c_prev + ig * gg
  h_new = og * jnp.tanh(c_new)
  return h_new, c_new


def _scan_body(ef_ref, eb_ref, wf_ref, uf_ref, bf_ref, wb_ref, ub_ref, bb_ref,
               hf_out, hb_out, hf_s, cf_s, hb_s, cb_s):
  i = pl.program_id(0)

  @pl.when(i == 0)
  def _():
    hf_s[...] = jnp.zeros_like(hf_s)
    cf_s[...] = jnp.zeros_like(cf_s)
    hb_s[...] = jnp.zeros_like(hb_s)
    cb_s[...] = jnp.zeros_like(cb_s)

  hf, cf = _lstm_step(ef_ref[0], hf_s[...], cf_s[...], wf_ref, uf_ref, bf_ref)
  hb, cb = _lstm_step(eb_ref[0], hb_s[...], cb_s[...], wb_ref, ub_ref, bb_ref)
  hf_s[...] = hf
  cf_s[...] = cf
  hb_s[...] = hb
  cb_s[...] = cb
  hf_out[0] = hf[:, :H]
  hb_out[0] = hb[:, :H]


def _pool_body(hf_ref, hb_ref, fcwf_ref, fcwb_ref, fcb_ref, out_ref,
               mxf, smf, mxb, smb):
  i = pl.program_id(0)

  @pl.when(i == 0)
  def _():
    mxf[...] = jnp.full_like(mxf, -jnp.inf)
    smf[...] = jnp.zeros_like(smf)
    mxb[...] = jnp.full_like(mxb, -jnp.inf)
    smb[...] = jnp.zeros_like(smb)

  hf = hf_ref[0]
  hb = hb_ref[0]
  s1 = jnp.sum(hf, axis=1, keepdims=True) + jnp.sum(hb, axis=1, keepdims=True)
  s2 = (jnp.sum(hf * hf, axis=1, keepdims=True)
        + jnp.sum(hb * hb, axis=1, keepdims=True))
  mu = s1 * (1.0 / (2 * H))
  var = s2 * (1.0 / (2 * H)) - mu * mu
  rstd = lax.rsqrt(var + 1e-5)
  lnf = (hf - mu) * rstd
  lnb = (hb - mu) * rstd
  mxf[...] = jnp.maximum(mxf[...], lnf)
  smf[...] = smf[...] + lnf
  mxb[...] = jnp.maximum(mxb[...], lnb)
  smb[...] = smb[...] + lnb

  @pl.when(i == L - 1)
  def _():
    zf = 0.5 * mxf[...] + (0.5 / L) * smf[...]
    zb = 0.5 * mxb[...] + (0.5 / L) * smb[...]
    logits = (jnp.dot(zf, fcwf_ref[...], preferred_element_type=F32)
              + jnp.dot(zb, fcwb_ref[...], preferred_element_type=F32)
              + fcb_ref[...])
    m = jnp.max(logits, axis=1, keepdims=True)
    lse = jnp.log(jnp.sum(jnp.exp(logits - m), axis=1, keepdims=True)) + m
    out_ref[...] = logits - lse


def _prep_gates(Wih, Whh, bih, bhh):
  wihT = Wih.T
  whhT = Whh.T
  bb = bih + bhh
  wih = jnp.zeros((EP, G4), F32)
  whh = jnp.zeros((HP, G4), F32)
  b = jnp.zeros((1, G4), F32)
  for g in range(4):
    wih = wih.at[:E, g * HP:g * HP + H].set(wihT[:, g * H:(g + 1) * H])
    whh = whh.at[:H, g * HP:g * HP + H].set(whhT[:, g * H:(g + 1) * H])
    b = b.at[0, g * HP:g * HP + H].set(bb[g * H:(g + 1) * H])
  return wih, whh, b


_FULL = lambda shape: pl.BlockSpec(shape, lambda i: tuple(0 for _ in shape))


def kernel(x, embed, Wih_f, Whh_f, bih_f, bhh_f, Wih_b, Whh_b, bih_b, bhh_b,
           fc_W, fc_b):
  idx = x.T.reshape(-1).astype(jnp.int32)
  embed_p = jnp.pad(embed.astype(F32), ((0, 0), (0, EP - E)))
  e = _make_gather()(embed_p, idx).reshape(L, B, EP)

  wih_b_, whh_b_, b_b_ = _prep_gates(Wih_b, Whh_b, bih_b, bhh_b)
  wih_f_, whh_f_, b_f_ = _prep_gates(Wih_f, Whh_f, bih_f, bhh_f)

  h_f, h_b = pl.pallas_call(
      _scan_body,
      grid=(L,),
      in_specs=[
          pl.BlockSpec((1, B, EP), lambda i: (i, 0, 0)),
          pl.BlockSpec((1, B, EP), lambda i: (L - 1 - i, 0, 0)),
          _FULL((EP, G4)),
          _FULL((HP, G4)),
          _FULL((1, G4)),
          _FULL((EP, G4)),
          _FULL((HP, G4)),
          _FULL((1, G4)),
      ],
      out_specs=[
          pl.BlockSpec((1, B, H), lambda i: (i, 0, 0)),
          pl.BlockSpec((1, B, H), lambda i: (L - 1 - i, 0, 0)),
      ],
      out_shape=[
          jax.ShapeDtypeStruct((L, B, H), F32),
          jax.ShapeDtypeStruct((L, B, H), F32),
      ],
      scratch_shapes=[pltpu.VMEM((B, HP), F32), pltpu.VMEM((B, HP), F32),
                      pltpu.VMEM((B, HP), F32), pltpu.VMEM((B, HP), F32)],
  )(e, e, wih_f_, whh_f_, b_f_, wih_b_, whh_b_, b_b_)

  fcwf = fc_W[:, :H].T
  fcwb = fc_W[:, H:].T
  fcb = fc_b.reshape(1, NCLS)

  out = pl.pallas_call(
      _pool_body,
      grid=(L,),
      in_specs=[
          pl.BlockSpec((1, B, H), lambda i: (i, 0, 0)),
          pl.BlockSpec((1, B, H), lambda i: (i, 0, 0)),
          _FULL((H, NCLS)),
          _FULL((H, NCLS)),
          _FULL((1, NCLS)),
      ],
      out_specs=pl.BlockSpec((B, NCLS), lambda i: (0, 0)),
      out_shape=jax.ShapeDtypeStruct((B, NCLS), F32),
      scratch_shapes=[pltpu.VMEM((B, H), F32), pltpu.VMEM((B, H), F32),
                      pltpu.VMEM((B, H), F32), pltpu.VMEM((B, H), F32)],
  )(h_f, h_b, fcwf, fcwb, fcb)

  return out


# speedup vs baseline: 2.2369x; 2.2369x over previous
import functools

import jax
import jax.numpy as jnp
from jax import lax
from jax.experimental import pallas as pl
from jax.experimental.pallas import tpu as pltpu
from jax.experimental.pallas import tpu_sc as plsc

B = 1024
L = 200
E = 64
EP = 128
H = 80
HP = 128
G4 = 4 * HP
NCLS = 15
F32 = jnp.float32


@functools.lru_cache(maxsize=None)
def _make_gather():
  info = plsc.get_sparse_core_info()
  nw = info.num_cores * info.num_subcores
  tot = B * L
  per_w = tot // nw
  ch = 800
  n_ch = per_w // ch

  @functools.partial(
      pl.kernel,
      out_type=jax.ShapeDtypeStruct((tot, EP), F32),
      mesh=plsc.VectorSubcoreMesh(core_axis_name="c", subcore_axis_name="s"),
      scratch_types=[
          pltpu.VMEM((per_w,), jnp.int32),
          pltpu.VMEM((ch, E), F32),
          pltpu.VMEM((ch, E), F32),
          pltpu.SemaphoreType.DMA,
          pltpu.SemaphoreType.DMA,
          pltpu.SemaphoreType.DMA,
          pltpu.SemaphoreType.DMA,
      ],
      compiler_params=pltpu.CompilerParams(use_tc_tiling_on_sc=False),
  )
  def gather_k(table_hbm, idx_hbm, out_hbm, idx_v, buf0, buf1, g0, g1, o0, o1):
    wid = lax.axis_index("s") * info.num_cores + lax.axis_index("c")
    base = wid * per_w
    pltpu.sync_copy(idx_hbm.at[pl.ds(base, per_w)], idx_v)
    bufs, gs, os = (buf0, buf1), (g0, g1), (o0, o1)
    gcopies = [pltpu.async_copy(table_hbm.at[idx_v.at[pl.ds(0, ch)]],
                                bufs[0], gs[0])]
    ocopies = []
    for j in range(1, n_ch + 1):
      if j < n_ch:
        if j >= 2:
          ocopies[j - 2].wait()
        gcopies.append(pltpu.async_copy(
            table_hbm.at[idx_v.at[pl.ds(j * ch, ch)]], bufs[j % 2], gs[j % 2]))
      gcopies[j - 1].wait()
      ocopies.append(pltpu.async_copy(
          bufs[(j - 1) % 2],
          out_hbm.at[pl.ds(base + (j - 1) * ch, ch), pl.ds(0, E)],
          os[(j - 1) % 2]))
    ocopies[n_ch - 2].wait()
    ocopies[n_ch - 1].wait()

  return gather_k


GT = 4 * H
UN = 5


HA = H + 8


def _aug(h_new):
  return jnp.concatenate([h_new, jnp.ones((HA - H, B), F32)], axis=0)


def _lstm_step(x_t, hT_prev, cT_prev, w_ref, u_ref):
  g = lax.dot_general(w_ref[...], x_t, (((1,), (1,)), ((), ())),
                      preferred_element_type=F32)
  g = g + lax.dot_general(u_ref[...], hT_prev, (((1,), (0,)), ((), ())),
                          preferred_element_type=F32)
  ig = 0.5 * jnp.tanh(g[0:H]) + 0.5
  fg = 0.5 * jnp.tanh(g[H:2 * H]) + 0.5
  gg = jnp.tanh(g[2 * H:3 * H])
  og = 0.5 * jnp.tanh(g[3 * H:4 * H]) + 0.5
  c_new = fg * cT_prev + ig * gg
  h_new = og * jnp.tanh(c_new)
  return h_new, c_new


HL = L // 2


def _ln_pair(hfv, hbv):
  s1 = jnp.sum(hfv, axis=0, keepdims=True) + jnp.sum(hbv, axis=0, keepdims=True)
  s2 = (jnp.sum(hfv * hfv, axis=0, keepdims=True)
        + jnp.sum(hbv * hbv, axis=0, keepdims=True))
  mu = s1 * (1.0 / (2 * H))
  var = s2 * (1.0 / (2 * H)) - mu * mu
  rstd = lax.rsqrt(var + 1e-5)
  return (hfv - mu) * rstd, (hbv - mu) * rstd


def _scan_pool_body(ef_ref, eb_ref, wf_ref, uf_ref, wb_ref, ub_ref,
                    fcwf_ref, fcwb_ref, fcb_ref, out_ref,
                    hf_s, cf_s, hb_s, cb_s, hfbuf, hbbuf, mxf, smf, mxb, smb):
  j = pl.program_id(0)

  @pl.when(j == 0)
  def _():
    hf_s[...] = _aug(jnp.zeros((H, B), F32))
    cf_s[...] = jnp.zeros_like(cf_s)
    hb_s[...] = _aug(jnp.zeros((H, B), F32))
    cb_s[...] = jnp.zeros_like(cb_s)
    mxf[...] = jnp.full_like(mxf, -jnp.inf)
    smf[...] = jnp.zeros_like(smf)
    mxb[...] = jnp.full_like(mxb, -jnp.inf)
    smb[...] = jnp.zeros_like(smb)

  hfs, hbs = [], []
  hfa, cf = hf_s[...], cf_s[...]
  hba, cb = hb_s[...], cb_s[...]
  for k in range(UN):
    hf, cf = _lstm_step(ef_ref[k][:, :E], hfa, cf, wf_ref, uf_ref)
    hb, cb = _lstm_step(eb_ref[UN - 1 - k][:, :E], hba, cb, wb_ref, ub_ref)
    hfa = _aug(hf)
    hba = _aug(hb)
    hfs.append(hf)
    hbs.append(hb)
  hf_s[...] = hfa
  cf_s[...] = cf
  hb_s[...] = hba
  cb_s[...] = cb

  @pl.when(j < HL // UN)
  def _():
    for k in range(UN):
      hfbuf[pl.ds(UN * j + k, 1)] = hfs[k].astype(jnp.bfloat16)[None]
      hbbuf[pl.ds(HL - 1 - UN * j - k, 1)] = hbs[k].astype(jnp.bfloat16)[None]

  @pl.when(j >= HL // UN)
  def _():
    mxfv, smfv, mxbv, smbv = mxf[...], smf[...], mxb[...], smb[...]
    for k in range(UN):
      hbA = hbbuf[pl.ds(UN * j + k - HL, 1)][0].astype(F32)
      hfB = hfbuf[pl.ds(L - 1 - UN * j - k, 1)][0].astype(F32)
      lnfA, lnbA = _ln_pair(hfs[k], hbA)
      lnfB, lnbB = _ln_pair(hfB, hbs[k])
      mxfv = jnp.maximum(mxfv, jnp.maximum(lnfA, lnfB))
      smfv = smfv + lnfA + lnfB
      mxbv = jnp.maximum(mxbv, jnp.maximum(lnbA, lnbB))
      smbv = smbv + lnbA + lnbB
    mxf[...] = mxfv
    smf[...] = smfv
    mxb[...] = mxbv
    smb[...] = smbv

  @pl.when(j == L // UN - 1)
  def _():
    zf = 0.5 * mxf[...] + (0.5 / L) * smf[...]
    zb = 0.5 * mxb[...] + (0.5 / L) * smb[...]
    logitsT = (jnp.dot(fcwf_ref[...], zf, preferred_element_type=F32)
               + jnp.dot(fcwb_ref[...], zb, preferred_element_type=F32)
               + fcb_ref[...])
    m = jnp.max(logitsT, axis=0, keepdims=True)
    lse = jnp.log(jnp.sum(jnp.exp(logitsT - m), axis=0, keepdims=True)) + m
    out_ref[...] = (logitsT - lse).T


def _prep_gates(Wih, Whh, bih, bhh):
  scale = jnp.concatenate([jnp.full((2 * H,), 0.5), jnp.ones((H,)),
                           jnp.full((H,), 0.5)]).astype(F32)
  w = Wih * scale[:, None]
  u = jnp.zeros((GT, HA), F32).at[:, :H].set(Whh * scale[:, None])
  u = u.at[:, H].set((bih + bhh) * scale)
  return w, u


_FULL = lambda shape: pl.BlockSpec(shape, lambda i: tuple(0 for _ in shape))


def kernel(x, embed, Wih_f, Whh_f, bih_f, bhh_f, Wih_b, Whh_b, bih_b, bhh_b,
           fc_W, fc_b):
  idx = x.T.reshape(-1).astype(jnp.int32)
  e = _make_gather()(embed.astype(F32), idx).reshape(L, B, EP)

  wf, uf = _prep_gates(Wih_f, Whh_f, bih_f, bhh_f)
  wb, ub = _prep_gates(Wih_b, Whh_b, bih_b, bhh_b)

  fcwf = fc_W[:, :H]
  fcwb = fc_W[:, H:]
  fcb = fc_b.reshape(NCLS, 1)

  out = pl.pallas_call(
      _scan_pool_body,
      grid=(L // UN,),
      in_specs=[
          pl.BlockSpec((UN, B, EP), lambda j: (j, 0, 0)),
          pl.BlockSpec((UN, B, EP), lambda j: (L // UN - 1 - j, 0, 0)),
          _FULL((GT, E)),
          _FULL((GT, HA)),
          _FULL((GT, E)),
          _FULL((GT, HA)),
          _FULL((NCLS, H)),
          _FULL((NCLS, H)),
          _FULL((NCLS, 1)),
      ],
      out_specs=pl.BlockSpec((B, NCLS), lambda j: (0, 0)),
      out_shape=jax.ShapeDtypeStruct((B, NCLS), F32),
      scratch_shapes=[
          pltpu.VMEM((HA, B), F32), pltpu.VMEM((H, B), F32),
          pltpu.VMEM((HA, B), F32), pltpu.VMEM((H, B), F32),
          pltpu.VMEM((HL, H, B), jnp.bfloat16),
          pltpu.VMEM((HL, H, B), jnp.bfloat16),
          pltpu.VMEM((H, B), F32), pltpu.VMEM((H, B), F32),
          pltpu.VMEM((H, B), F32), pltpu.VMEM((H, B), F32),
      ],
  )(e, e, wf, uf, wb, ub, fcwf, fcwb, fcb)

  return out
